# baseline (speedup 1.0000x reference)
import functools

import jax
import jax.numpy as jnp
from jax import lax
from jax.experimental import pallas as pl
from jax.experimental.pallas import tpu as pltpu
from jax.experimental.pallas import tpu_sc as plsc

_NC = 2
_NS = 16
_NW = _NC * _NS

_N_IMG = 4096
_K = 50
_D = 128
_IPW = _N_IMG // _NW
_NBUF = 4


@functools.partial(
    pl.kernel,
    out_type=jax.ShapeDtypeStruct((_N_IMG, _K, _D), jnp.float32),
    mesh=plsc.VectorSubcoreMesh(
        core_axis_name="c", subcore_axis_name="s",
        num_cores=_NC, num_subcores=_NS),
    compiler_params=pltpu.CompilerParams(use_tc_tiling_on_sc=True),
    scratch_types=[
        pltpu.VMEM((_IPW, _K), jnp.int32),
        pltpu.VMEM((_NBUF, _K, _D), jnp.float32),
        pltpu.SemaphoreType.DMA,
        pltpu.SemaphoreType.DMA,
    ],
)
def _gather_kernel(table_hbm, idx_hbm, out_hbm, idx_v, rows_v, gsem, wsem):
    wid = lax.axis_index("s") * _NC + lax.axis_index("c")
    img0 = wid * _IPW
    pltpu.sync_copy(idx_hbm.at[pl.ds(img0, _IPW)], idx_v)

    def g_desc(j, b):
        return pltpu.make_async_copy(
            table_hbm.at[idx_v.at[j]], rows_v.at[b], gsem)

    def w_desc(j, b):
        return pltpu.make_async_copy(
            rows_v.at[b], out_hbm.at[img0 + j], wsem)

    g_desc(0, 0).start()
    g_desc(1, 1).start()
    for jj in (0, 1):
        g_desc(jj, jj).wait()
        w_desc(jj, jj).start()
        g_desc(jj + 2, jj + 2).start()

    @pl.loop(2, _IPW - 2, step=_NBUF)
    def _steady(j):
        for i in range(_NBUF):
            jj = j + i
            b = (2 + i) % _NBUF
            bn = (b + 2) % _NBUF
            g_desc(jj, b).wait()
            w_desc(jj, b).start()
            w_desc(jj - 2, bn).wait()
            g_desc(jj + 2, bn).start()

    for jj in (_IPW - 2, _IPW - 1):
        b = jj % _NBUF
        g_desc(jj, b).wait()
        w_desc(jj, b).start()
        w_desc(jj - 2, (b + 2) % _NBUF).wait()
    w_desc(_IPW - 2, (_IPW - 2) % _NBUF).wait()
    w_desc(_IPW - 1, (_IPW - 1) % _NBUF).wait()


def kernel(class_embed_weight, indices):
    return _gather_kernel(class_embed_weight, indices.astype(jnp.int32))


# speedup vs baseline: 1.9242x; 1.9242x over previous
import functools

import jax
import jax.numpy as jnp
from jax import lax
from jax.experimental import pallas as pl
from jax.experimental.pallas import tpu as pltpu
from jax.experimental.pallas import tpu_sc as plsc

_NC = 2
_NS = 16
_NW = _NC * _NS

_B = 4096 * 50
_D = 128
_BPW = _B // _NW
_CHUNK = 128
_NCHUNK = _BPW // _CHUNK
_NBUF = 4


@functools.partial(
    pl.kernel,
    out_type=jax.ShapeDtypeStruct((_B, _D), jnp.float32),
    mesh=plsc.VectorSubcoreMesh(
        core_axis_name="c", subcore_axis_name="s",
        num_cores=_NC, num_subcores=_NS),
    scratch_types=[
        pltpu.VMEM((_BPW,), jnp.int32),
        pltpu.VMEM((_NBUF, _CHUNK, _D), jnp.float32),
        pltpu.SemaphoreType.DMA,
        pltpu.SemaphoreType.DMA,
    ],
)
def _gather_kernel(table_hbm, idx_hbm, out_hbm, idx_v, rows_v, gsem, wsem):
    wid = lax.axis_index("s") * _NC + lax.axis_index("c")
    base = wid * _BPW
    pltpu.sync_copy(idx_hbm.at[pl.ds(base, _BPW)], idx_v)

    def g_desc(j, b):
        off = pl.multiple_of(j * _CHUNK, 8)
        return pltpu.make_async_copy(
            table_hbm.at[idx_v.at[pl.ds(off, _CHUNK)]], rows_v.at[b], gsem)

    def w_desc(j, b):
        off = pl.multiple_of(j * _CHUNK, 8)
        return pltpu.make_async_copy(
            rows_v.at[b], out_hbm.at[pl.ds(base + off, _CHUNK)], wsem)

    g_desc(0, 0).start()
    g_desc(1, 1).start()
    for jj in (0, 1):
        g_desc(jj, jj).wait()
        w_desc(jj, jj).start()
        g_desc(jj + 2, jj + 2).start()

    @pl.loop(2, _NCHUNK - 4, step=_NBUF)
    def _steady(j):
        for i in range(_NBUF):
            jj = j + i
            b = (2 + i) % _NBUF
            bn = (b + 2) % _NBUF
            g_desc(jj, b).wait()
            w_desc(jj, b).start()
            w_desc(jj - 2, bn).wait()
            g_desc(jj + 2, bn).start()

    for jj in (_NCHUNK - 4, _NCHUNK - 3):
        b = jj % _NBUF
        bn = (b + 2) % _NBUF
        g_desc(jj, b).wait()
        w_desc(jj, b).start()
        w_desc(jj - 2, bn).wait()
        g_desc(jj + 2, bn).start()
    for jj in (_NCHUNK - 2, _NCHUNK - 1):
        b = jj % _NBUF
        g_desc(jj, b).wait()
        w_desc(jj, b).start()
        w_desc(jj - 2, (b + 2) % _NBUF).wait()
    w_desc(_NCHUNK - 2, (_NCHUNK - 2) % _NBUF).wait()
    w_desc(_NCHUNK - 1, (_NCHUNK - 1) % _NBUF).wait()


def kernel(class_embed_weight, indices):
    n_img, k = indices.shape
    idx_flat = jnp.transpose(indices).reshape(-1).astype(jnp.int32)
    out_flat = _gather_kernel(class_embed_weight, idx_flat)
    return out_flat.reshape(k, n_img, _D).transpose(1, 0, 2)


# speedup vs baseline: 1.9347x; 1.0055x over previous
import functools

import jax
import jax.numpy as jnp
from jax import lax
from jax.experimental import pallas as pl
from jax.experimental.pallas import tpu as pltpu
from jax.experimental.pallas import tpu_sc as plsc

_NC = 2
_NS = 16
_NW = _NC * _NS

_B = 4096 * 50
_D = 128
_BPW = _B // _NW
_CHUNK = 160
_NCHUNK = _BPW // _CHUNK
_NBUF = 4
_LOOP_END = 2 + _NBUF * ((_NCHUNK - 4) // _NBUF)


@functools.partial(
    pl.kernel,
    out_type=jax.ShapeDtypeStruct((_B, _D), jnp.float32),
    mesh=plsc.VectorSubcoreMesh(
        core_axis_name="c", subcore_axis_name="s",
        num_cores=_NC, num_subcores=_NS),
    scratch_types=[
        pltpu.VMEM((_BPW,), jnp.int32),
        pltpu.VMEM((_NBUF, _CHUNK, _D), jnp.float32),
        pltpu.SemaphoreType.DMA,
        pltpu.SemaphoreType.DMA,
    ],
)
def _gather_kernel(table_hbm, idx_hbm, out_hbm, idx_v, rows_v, gsem, wsem):
    wid = lax.axis_index("s") * _NC + lax.axis_index("c")
    base = wid * _BPW
    pltpu.sync_copy(idx_hbm.at[pl.ds(base, _BPW)], idx_v)

    def g_desc(j, b):
        off = pl.multiple_of(j * _CHUNK, 8)
        return pltpu.make_async_copy(
            table_hbm.at[idx_v.at[pl.ds(off, _CHUNK)]], rows_v.at[b], gsem)

    def w_desc(j, b):
        off = pl.multiple_of(j * _CHUNK, 8)
        return pltpu.make_async_copy(
            rows_v.at[b], out_hbm.at[pl.ds(base + off, _CHUNK)], wsem)

    g_desc(0, 0).start()
    g_desc(1, 1).start()
    for jj in (0, 1):
        g_desc(jj, jj).wait()
        w_desc(jj, jj).start()
        g_desc(jj + 2, jj + 2).start()

    @pl.loop(2, _LOOP_END, step=_NBUF)
    def _steady(j):
        for i in range(_NBUF):
            jj = j + i
            b = (2 + i) % _NBUF
            bn = (b + 2) % _NBUF
            g_desc(jj, b).wait()
            w_desc(jj, b).start()
            w_desc(jj - 2, bn).wait()
            g_desc(jj + 2, bn).start()

    for jj in range(_LOOP_END, _NCHUNK - 2):
        b = jj % _NBUF
        bn = (b + 2) % _NBUF
        g_desc(jj, b).wait()
        w_desc(jj, b).start()
        w_desc(jj - 2, bn).wait()
        g_desc(jj + 2, bn).start()
    for jj in (_NCHUNK - 2, _NCHUNK - 1):
        b = jj % _NBUF
        g_desc(jj, b).wait()
        w_desc(jj, b).start()
        w_desc(jj - 2, (b + 2) % _NBUF).wait()
    w_desc(_NCHUNK - 2, (_NCHUNK - 2) % _NBUF).wait()
    w_desc(_NCHUNK - 1, (_NCHUNK - 1) % _NBUF).wait()


def kernel(class_embed_weight, indices):
    n_img, k = indices.shape
    idx_flat = jnp.transpose(indices).reshape(-1).astype(jnp.int32)
    out_flat = _gather_kernel(class_embed_weight, idx_flat)
    return out_flat.reshape(k, n_img, _D).transpose(1, 0, 2)


# speedup vs baseline: 1.9492x; 1.0075x over previous
import functools

import jax
import jax.numpy as jnp
from jax import lax
from jax.experimental import pallas as pl
from jax.experimental.pallas import tpu as pltpu
from jax.experimental.pallas import tpu_sc as plsc

_NC = 2
_NS = 16
_NW = _NC * _NS

_B = 4096 * 50
_D = 128
_BPW = _B // _NW
_CHUNK = 160
_NCHUNK = _BPW // _CHUNK
_LEAD = 3
_NBUF = 2 * _LEAD
_LOOP_END = _LEAD + _NBUF * ((_NCHUNK - 2 * _LEAD) // _NBUF)


@functools.partial(
    pl.kernel,
    out_type=jax.ShapeDtypeStruct((_B, _D), jnp.float32),
    mesh=plsc.VectorSubcoreMesh(
        core_axis_name="c", subcore_axis_name="s",
        num_cores=_NC, num_subcores=_NS),
    scratch_types=[
        pltpu.VMEM((_BPW,), jnp.int32),
        pltpu.VMEM((_NBUF, _CHUNK, _D), jnp.float32),
        pltpu.SemaphoreType.DMA,
        pltpu.SemaphoreType.DMA,
    ],
)
def _gather_kernel(table_hbm, idx_hbm, out_hbm, idx_v, rows_v, gsem, wsem):
    wid = lax.axis_index("s") * _NC + lax.axis_index("c")
    base = wid * _BPW
    pltpu.sync_copy(idx_hbm.at[pl.ds(base, _BPW)], idx_v)

    def g_desc(j, b):
        off = pl.multiple_of(j * _CHUNK, 8)
        return pltpu.make_async_copy(
            table_hbm.at[idx_v.at[pl.ds(off, _CHUNK)]], rows_v.at[b], gsem)

    def w_desc(j, b):
        off = pl.multiple_of(j * _CHUNK, 8)
        return pltpu.make_async_copy(
            rows_v.at[b], out_hbm.at[pl.ds(base + off, _CHUNK)], wsem)

    for jj in range(_LEAD):
        g_desc(jj, jj).start()
    for jj in range(_LEAD):
        g_desc(jj, jj).wait()
        w_desc(jj, jj).start()
        g_desc(jj + _LEAD, jj + _LEAD).start()

    @pl.loop(_LEAD, _LOOP_END, step=_NBUF)
    def _steady(j):
        for i in range(_NBUF):
            jj = j + i
            b = (_LEAD + i) % _NBUF
            bn = (b + _LEAD) % _NBUF
            g_desc(jj, b).wait()
            w_desc(jj, b).start()
            w_desc(jj - _LEAD, bn).wait()
            g_desc(jj + _LEAD, bn).start()

    for jj in range(_LOOP_END, _NCHUNK - _LEAD):
        b = jj % _NBUF
        bn = (b + _LEAD) % _NBUF
        g_desc(jj, b).wait()
        w_desc(jj, b).start()
        w_desc(jj - _LEAD, bn).wait()
        g_desc(jj + _LEAD, bn).start()
    for jj in range(_NCHUNK - _LEAD, _NCHUNK):
        b = jj % _NBUF
        g_desc(jj, b).wait()
        w_desc(jj, b).start()
        w_desc(jj - _LEAD, (b + _LEAD) % _NBUF).wait()
    for jj in range(_NCHUNK - _LEAD, _NCHUNK):
        w_desc(jj, jj % _NBUF).wait()


def kernel(class_embed_weight, indices):
    n_img, k = indices.shape
    idx_flat = jnp.transpose(indices).reshape(-1).astype(jnp.int32)
    out_flat = _gather_kernel(class_embed_weight, idx_flat)
    return out_flat.reshape(k, n_img, _D).transpose(1, 0, 2)


# speedup vs baseline: 1.9496x; 1.0002x over previous
import functools

import jax
import jax.numpy as jnp
from jax import lax
from jax.experimental import pallas as pl
from jax.experimental.pallas import tpu as pltpu
from jax.experimental.pallas import tpu_sc as plsc

_NC = 2
_NS = 16
_NW = _NC * _NS

_B = 4096 * 50
_D = 128
_BPW = _B // _NW
_CHUNK = 160
_NCHUNK = _BPW // _CHUNK
_LEAD = 3
_NBUF = 2 * _LEAD
_LOOP_END = _LEAD + _NBUF * ((_NCHUNK - 2 * _LEAD) // _NBUF)


@functools.partial(
    pl.kernel,
    out_type=jax.ShapeDtypeStruct((_B, _D), jnp.float32),
    mesh=plsc.VectorSubcoreMesh(
        core_axis_name="c", subcore_axis_name="s",
        num_cores=_NC, num_subcores=_NS),
    compiler_params=pltpu.CompilerParams(skip_device_barrier=True),
    scratch_types=[
        pltpu.VMEM((_BPW,), jnp.int32),
        pltpu.VMEM((_NBUF, _CHUNK, _D), jnp.float32),
        pltpu.SemaphoreType.DMA,
        pltpu.SemaphoreType.DMA,
    ],
)
def _gather_kernel(table_hbm, idx_hbm, out_hbm, idx_v, rows_v, gsem, wsem):
    wid = lax.axis_index("s") * _NC + lax.axis_index("c")
    base = wid * _BPW
    pltpu.sync_copy(idx_hbm.at[pl.ds(base, _BPW)], idx_v)

    def g_desc(j, b):
        off = pl.multiple_of(j * _CHUNK, 8)
        return pltpu.make_async_copy(
            table_hbm.at[idx_v.at[pl.ds(off, _CHUNK)]], rows_v.at[b], gsem)

    def w_desc(j, b):
        off = pl.multiple_of(j * _CHUNK, 8)
        return pltpu.make_async_copy(
            rows_v.at[b], out_hbm.at[pl.ds(base + off, _CHUNK)], wsem)

    for jj in range(_LEAD):
        g_desc(jj, jj).start()
    for jj in range(_LEAD):
        g_desc(jj, jj).wait()
        w_desc(jj, jj).start()
        g_desc(jj + _LEAD, jj + _LEAD).start()

    @pl.loop(_LEAD, _LOOP_END, step=_NBUF)
    def _steady(j):
        for i in range(_NBUF):
            jj = j + i
            b = (_LEAD + i) % _NBUF
            bn = (b + _LEAD) % _NBUF
            g_desc(jj, b).wait()
            w_desc(jj, b).start()
            w_desc(jj - _LEAD, bn).wait()
            g_desc(jj + _LEAD, bn).start()

    for jj in range(_LOOP_END, _NCHUNK - _LEAD):
        b = jj % _NBUF
        bn = (b + _LEAD) % _NBUF
        g_desc(jj, b).wait()
        w_desc(jj, b).start()
        w_desc(jj - _LEAD, bn).wait()
        g_desc(jj + _LEAD, bn).start()
    for jj in range(_NCHUNK - _LEAD, _NCHUNK):
        b = jj % _NBUF
        g_desc(jj, b).wait()
        w_desc(jj, b).start()
        w_desc(jj - _LEAD, (b + _LEAD) % _NBUF).wait()
    for jj in range(_NCHUNK - _LEAD, _NCHUNK):
        w_desc(jj, jj % _NBUF).wait()


def kernel(class_embed_weight, indices):
    n_img, k = indices.shape
    idx_flat = jnp.transpose(indices).reshape(-1).astype(jnp.int32)
    out_flat = _gather_kernel(class_embed_weight, idx_flat)
    return out_flat.reshape(k, n_img, _D).transpose(1, 0, 2)
